# initial kernel scaffold (unmeasured)
import jax
import jax.numpy as jnp
from jax import lax
from jax.experimental import pallas as pl
from jax.experimental.pallas import tpu as pltpu


def kernel(
    x,
):
    def body(*refs):
        pass

    out_shape = jax.ShapeDtypeStruct(..., jnp.float32)
    return pl.pallas_call(body, out_shape=out_shape)(...)


# baseline (device time: 19221 ns/iter reference)
import jax
import jax.numpy as jnp
from jax import lax
from jax.experimental import pallas as pl
from jax.experimental.pallas import tpu as pltpu

N_DEV = 4
HOPS = N_DEV - 1


def kernel(x):
    m_per, n = x.shape
    half = m_per // 2

    def body(x_ref, out_ref, send_cw, recv_cw, send_ccw, recv_ccw):
        me = lax.axis_index("i")
        right = lax.rem(me + 1, N_DEV)
        left = lax.rem(me + N_DEV - 1, N_DEV)

        barrier = pltpu.get_barrier_semaphore()
        for nbr in (left, right):
            pl.semaphore_signal(
                barrier, inc=1,
                device_id=(nbr,), device_id_type=pl.DeviceIdType.MESH,
            )
        pl.semaphore_wait(barrier, 2)

        out_ref[pl.ds(me * m_per, m_per), :] = x_ref[:, :].astype(out_ref.dtype)

        def cw_slice(origin):
            return out_ref.at[pl.ds(origin * m_per, half), :]

        def ccw_slice(origin):
            return out_ref.at[pl.ds(origin * m_per + half, half), :]

        sends = []
        recvs = []
        for h in range(HOPS):
            o_s_cw = lax.rem(me + N_DEV - h, N_DEV)
            o_r_cw = lax.rem(me + N_DEV - h - 1, N_DEV)
            o_s_ccw = lax.rem(me + h, N_DEV)
            o_r_ccw = lax.rem(me + h + 1, N_DEV)
            s_cw = pltpu.make_async_remote_copy(
                src_ref=cw_slice(o_s_cw), dst_ref=cw_slice(o_s_cw),
                send_sem=send_cw.at[h], recv_sem=recv_cw.at[h],
                device_id=(right,), device_id_type=pl.DeviceIdType.MESH,
            )
            s_ccw = pltpu.make_async_remote_copy(
                src_ref=ccw_slice(o_s_ccw), dst_ref=ccw_slice(o_s_ccw),
                send_sem=send_ccw.at[h], recv_sem=recv_ccw.at[h],
                device_id=(left,), device_id_type=pl.DeviceIdType.MESH,
            )
            r_cw = pltpu.make_async_remote_copy(
                src_ref=cw_slice(o_r_cw), dst_ref=cw_slice(o_r_cw),
                send_sem=send_cw.at[h], recv_sem=recv_cw.at[h],
                device_id=(left,), device_id_type=pl.DeviceIdType.MESH,
            )
            r_ccw = pltpu.make_async_remote_copy(
                src_ref=ccw_slice(o_r_ccw), dst_ref=ccw_slice(o_r_ccw),
                send_sem=send_ccw.at[h], recv_sem=recv_ccw.at[h],
                device_id=(right,), device_id_type=pl.DeviceIdType.MESH,
            )
            sends.append((s_cw, s_ccw))
            recvs.append((r_cw, r_ccw))

        for h in range(HOPS):
            sends[h][0].start()
            sends[h][1].start()
            sends[h][0].wait_send()
            sends[h][1].wait_send()
            recvs[h][0].wait_recv()
            recvs[h][1].wait_recv()

    return pl.pallas_call(
        body,
        out_shape=jax.ShapeDtypeStruct((N_DEV * m_per, n), jnp.bfloat16),
        in_specs=[pl.BlockSpec(memory_space=pltpu.VMEM)],
        out_specs=pl.BlockSpec(memory_space=pltpu.VMEM),
        scratch_shapes=[
            pltpu.SemaphoreType.DMA((HOPS,)),
            pltpu.SemaphoreType.DMA((HOPS,)),
            pltpu.SemaphoreType.DMA((HOPS,)),
            pltpu.SemaphoreType.DMA((HOPS,)),
        ],
        compiler_params=pltpu.CompilerParams(collective_id=0),
    )(x)
